# speedup vs baseline: 4.1481x; 4.1481x over previous
import functools

import jax
import jax.numpy as jnp
from jax import lax
from jax.experimental import pallas as pl
from jax.experimental.pallas import tpu as pltpu
from jax.experimental.pallas import tpu_sc as plsc

ALPHA_BLEND = 0.3

C = 80
ZR = 128


def _sc_accumulate(src_hbm, sidx_hbm, didx_hbm, edge_hbm,
                   sum_out, cnt_out,
                   acc_sh, cnt_sh,
                   sidx_v, didx_v, srows_v, erows_v, ones_v, zbuf_v, zcnt_v,
                   sem):
    N, D = src_hbm.shape
    E = sidx_hbm.shape[0]
    npad = cnt_sh.shape[0]
    rows_per_tile = npad // 16
    epw = E // 32
    chunks = epw // C

    cid = lax.axis_index("c")
    sid = lax.axis_index("s")
    wid = sid * 2 + cid

    zero16 = jnp.zeros((16,), jnp.float32)
    one16 = jnp.ones((16,), jnp.float32)

    def zrow(i, carry):
        for j in range(8):
            zbuf_v[i, pl.ds(j * 16, 16)] = zero16
        return carry
    lax.fori_loop(0, ZR, zrow, 0)

    def zcnt(i, carry):
        zcnt_v[pl.ds(i * 16, 16)] = zero16
        return carry
    lax.fori_loop(0, rows_per_tile // 16, zcnt, 0)

    for j in range(C // 16):
        ones_v[pl.ds(j * 16, 16)] = one16

    def zacc(k, carry):
        pltpu.sync_copy(zbuf_v, acc_sh.at[pl.ds(sid * rows_per_tile + k * ZR, ZR)])
        return carry
    lax.fori_loop(0, rows_per_tile // ZR, zacc, 0)
    pltpu.sync_copy(zcnt_v, cnt_sh.at[pl.ds(sid * rows_per_tile, rows_per_tile)])

    plsc.subcore_barrier()

    ebase = wid * epw

    def body(j, carry):
        base = ebase + j * C
        pltpu.sync_copy(sidx_hbm.at[pl.ds(base, C)], sidx_v)
        pltpu.sync_copy(didx_hbm.at[pl.ds(base, C)], didx_v)
        pltpu.async_copy(src_hbm.at[sidx_v], srows_v, sem).wait()
        pltpu.sync_copy(edge_hbm.at[pl.ds(base, C)], erows_v)
        pltpu.sync_copy(srows_v, acc_sh.at[didx_v], add=True)
        pltpu.sync_copy(erows_v, acc_sh.at[didx_v], add=True)
        pltpu.sync_copy(ones_v, cnt_sh.at[didx_v], add=True)
        return carry
    lax.fori_loop(0, chunks, body, 0)

    plsc.subcore_barrier()

    pltpu.sync_copy(cnt_sh.at[pl.ds(sid * rows_per_tile, rows_per_tile)],
                    cnt_out.at[cid, pl.ds(sid * rows_per_tile, rows_per_tile)])

    last_base = 15 * rows_per_tile
    last_rows = N - last_base

    @pl.when(sid < 15)
    def _():
        pltpu.sync_copy(acc_sh.at[pl.ds(sid * rows_per_tile, rows_per_tile)],
                        sum_out.at[cid, pl.ds(sid * rows_per_tile, rows_per_tile)])

    @pl.when(sid == 15)
    def _():
        pltpu.sync_copy(acc_sh.at[pl.ds(last_base, last_rows)],
                        sum_out.at[cid, pl.ds(last_base, last_rows)])


def _tc_combine(sum_ref, cnt_ref, dst_ref, out_ref):
    N = dst_ref.shape[0]
    s = sum_ref[0, :N, :] + sum_ref[1, :N, :]
    c = cnt_ref[0, :N, :] + cnt_ref[1, :N, :]
    mean = s / jnp.maximum(c, 1.0)
    agg = ALPHA_BLEND * dst_ref[...] + (1.0 - ALPHA_BLEND) * mean
    out_ref[...] = jnp.where(c > 0.0, agg, 0.0)


def kernel(src_embedding, dst_embedding, edge_embedding, edge_index):
    N, D = src_embedding.shape
    E = edge_embedding.shape[0]
    npad = ((N + 639) // 640) * 640

    src_idx = edge_index[0].astype(jnp.int32)
    dst_idx = edge_index[1].astype(jnp.int32)

    mesh = plsc.VectorSubcoreMesh(core_axis_name="c", subcore_axis_name="s")
    sc_call = pl.kernel(
        _sc_accumulate,
        out_type=(
            jax.ShapeDtypeStruct((2, N, D), jnp.float32),
            jax.ShapeDtypeStruct((2, npad), jnp.float32),
        ),
        mesh=mesh,
        scratch_types=[
            pltpu.VMEM_SHARED((npad, D), jnp.float32),
            pltpu.VMEM_SHARED((npad,), jnp.float32),
            pltpu.VMEM((C,), jnp.int32),
            pltpu.VMEM((C,), jnp.int32),
            pltpu.VMEM((C, D), jnp.float32),
            pltpu.VMEM((C, D), jnp.float32),
            pltpu.VMEM((C,), jnp.float32),
            pltpu.VMEM((ZR, D), jnp.float32),
            pltpu.VMEM((npad // 16,), jnp.float32),
            pltpu.SemaphoreType.DMA,
        ],
    )
    sums, cnts = sc_call(src_embedding, src_idx, dst_idx, edge_embedding)

    cnts3 = cnts.reshape(2, npad, 1)
    out = pl.pallas_call(
        _tc_combine,
        out_shape=jax.ShapeDtypeStruct((N, D), jnp.float32),
    )(sums, cnts3, dst_embedding)
    return out


# speedup vs baseline: 9.5382x; 2.2994x over previous
import jax
import jax.numpy as jnp
from jax import lax
from jax.experimental import pallas as pl
from jax.experimental.pallas import tpu as pltpu
from jax.experimental.pallas import tpu_sc as plsc

ALPHA_BLEND = 0.3

C = 80
NBUF = 2
IB = 4


def _sc_accumulate(src_hbm, sidx_hbm, didx_hbm, edge_hbm,
                   sum_out, cnt_out,
                   acc_sh, cnt_sh,
                   sidx_r, didx_r, srows, erows, ones_v, zcnt_v,
                   idx_sem, load_sem, scat_sem):
    N, D = src_hbm.shape
    E = sidx_hbm.shape[0]
    epw = E // 32
    chunks = epw // C
    npad = cnt_sh.shape[0]
    rows_per_tile = npad // 16

    cid = lax.axis_index("c")
    sid = lax.axis_index("s")
    wid = sid * 2 + cid
    ebase = wid * epw

    zero16 = jnp.zeros((16,), jnp.float32)
    one16 = jnp.ones((16,), jnp.float32)

    def zrow(i, carry):
        for j in range(D // 16):
            srows[0, i, pl.ds(j * 16, 16)] = zero16
        return carry
    lax.fori_loop(0, C, zrow, 0)

    def zcnt(i, carry):
        zcnt_v[pl.ds(i * 16, 16)] = zero16
        return carry
    lax.fori_loop(0, rows_per_tile // 16, zcnt, 0)

    for j in range(C // 16):
        ones_v[pl.ds(j * 16, 16)] = one16

    def zacc(k, carry):
        pltpu.sync_copy(srows.at[0],
                        acc_sh.at[pl.ds(sid * rows_per_tile + k * C, C)])
        return carry
    lax.fori_loop(0, rows_per_tile // C, zacc, 0)
    pltpu.sync_copy(zcnt_v, cnt_sh.at[pl.ds(sid * rows_per_tile, rows_per_tile)])

    plsc.subcore_barrier()

    def issue_idx(j, s):
        pltpu.async_copy(sidx_hbm.at[pl.ds(ebase + j * C, C)], sidx_r.at[s],
                         idx_sem.at[s])
        pltpu.async_copy(didx_hbm.at[pl.ds(ebase + j * C, C)], didx_r.at[s],
                         idx_sem.at[s])

    def wait_idx(j, s):
        pltpu.make_async_copy(sidx_hbm.at[pl.ds(ebase + j * C, C)],
                              sidx_r.at[s], idx_sem.at[s]).wait()
        pltpu.make_async_copy(didx_hbm.at[pl.ds(ebase + j * C, C)],
                              didx_r.at[s], idx_sem.at[s]).wait()

    def issue_loads(j, s, b):
        pltpu.async_copy(src_hbm.at[sidx_r.at[s]], srows.at[b],
                         load_sem.at[b])
        pltpu.async_copy(edge_hbm.at[pl.ds(ebase + j * C, C)], erows.at[b],
                         load_sem.at[b])

    def wait_loads(j, s, b):
        pltpu.make_async_copy(src_hbm.at[sidx_r.at[s]], srows.at[b],
                              load_sem.at[b]).wait()
        pltpu.make_async_copy(edge_hbm.at[pl.ds(ebase + j * C, C)],
                              erows.at[b], load_sem.at[b]).wait()

    def issue_scatters(s, b):
        pltpu.async_copy(srows.at[b], acc_sh.at[didx_r.at[s]],
                         scat_sem.at[b], add=True)
        pltpu.async_copy(erows.at[b], acc_sh.at[didx_r.at[s]],
                         scat_sem.at[b], add=True)
        pltpu.async_copy(ones_v, cnt_sh.at[didx_r.at[s]],
                         scat_sem.at[b], add=True)

    def wait_scatters(s, b):
        pltpu.make_async_copy(srows.at[b], acc_sh.at[didx_r.at[s]],
                              scat_sem.at[b]).wait()
        pltpu.make_async_copy(erows.at[b], acc_sh.at[didx_r.at[s]],
                              scat_sem.at[b]).wait()
        pltpu.make_async_copy(ones_v, cnt_sh.at[didx_r.at[s]],
                              scat_sem.at[b]).wait()

    for p in range(IB - 1):
        issue_idx(p, p)
    wait_idx(0, 0)
    issue_loads(0, 0, 0)

    def group(g, carry):
        for b4 in range(IB):
            j = g * IB + b4
            b = b4 % NBUF
            pb = (b + 1) % NBUF
            si = b4 % IB
            sn = (b4 + 1) % IB
            sp = (b4 + IB - 1) % IB

            @pl.when(j >= 1)
            def _():
                wait_scatters(sp, pb)

            @pl.when(j + IB - 1 < chunks)
            def _():
                issue_idx(j + IB - 1, sp)

            @pl.when(j + 1 < chunks)
            def _():
                wait_idx(j + 1, sn)
                issue_loads(j + 1, sn, pb)

            wait_loads(j, si, b)
            issue_scatters(si, b)
        return carry
    lax.fori_loop(0, chunks // IB, group, 0)

    for j in range((chunks // IB) * IB, chunks):
        b, pb, si = j % NBUF, (j + 1) % NBUF, j % IB
        wait_scatters((si + IB - 1) % IB, pb)
        if j + 1 < chunks:
            wait_idx(j + 1, (si + 1) % IB)
            issue_loads(j + 1, (si + 1) % IB, pb)
        wait_loads(j, si, b)
        issue_scatters(si, b)
    wait_scatters((chunks - 1) % IB, (chunks - 1) % NBUF)

    plsc.subcore_barrier()

    pltpu.sync_copy(cnt_sh.at[pl.ds(sid * rows_per_tile, rows_per_tile)],
                    cnt_out.at[cid, pl.ds(sid * rows_per_tile, rows_per_tile)])

    last_base = 15 * rows_per_tile
    last_rows = N - last_base

    @pl.when(sid < 15)
    def _():
        pltpu.sync_copy(acc_sh.at[pl.ds(sid * rows_per_tile, rows_per_tile)],
                        sum_out.at[cid, pl.ds(sid * rows_per_tile, rows_per_tile)])

    @pl.when(sid == 15)
    def _():
        pltpu.sync_copy(acc_sh.at[pl.ds(last_base, last_rows)],
                        sum_out.at[cid, pl.ds(last_base, last_rows)])


def _tc_combine(sum_ref, cnt_ref, dst_ref, out_ref):
    N = dst_ref.shape[0]
    s = sum_ref[0, :N, :] + sum_ref[1, :N, :]
    c = cnt_ref[0, :N, :] + cnt_ref[1, :N, :]
    mean = s / jnp.maximum(c, 1.0)
    agg = ALPHA_BLEND * dst_ref[...] + (1.0 - ALPHA_BLEND) * mean
    out_ref[...] = jnp.where(c > 0.0, agg, 0.0)


def kernel(src_embedding, dst_embedding, edge_embedding, edge_index):
    N, D = src_embedding.shape
    E = edge_embedding.shape[0]
    npad = ((N + 639) // 640) * 640

    src_idx = edge_index[0].astype(jnp.int32)
    dst_idx = edge_index[1].astype(jnp.int32)

    mesh = plsc.VectorSubcoreMesh(core_axis_name="c", subcore_axis_name="s")
    sc_call = pl.kernel(
        _sc_accumulate,
        out_type=(
            jax.ShapeDtypeStruct((2, N, D), jnp.float32),
            jax.ShapeDtypeStruct((2, npad), jnp.float32),
        ),
        mesh=mesh,
        scratch_types=[
            pltpu.VMEM_SHARED((npad, D), jnp.float32),
            pltpu.VMEM_SHARED((npad,), jnp.float32),
            pltpu.VMEM((IB, C), jnp.int32),
            pltpu.VMEM((IB, C), jnp.int32),
            pltpu.VMEM((NBUF, C, D), jnp.float32),
            pltpu.VMEM((NBUF, C, D), jnp.float32),
            pltpu.VMEM((C,), jnp.float32),
            pltpu.VMEM((npad // 16,), jnp.float32),
            pltpu.SemaphoreType.DMA((IB,)),
            pltpu.SemaphoreType.DMA((NBUF,)),
            pltpu.SemaphoreType.DMA((NBUF,)),
        ],
    )
    sums, cnts = sc_call(src_embedding, src_idx, dst_idx, edge_embedding)

    cnts3 = cnts.reshape(2, npad, 1)
    out = pl.pallas_call(
        _tc_combine,
        out_shape=jax.ShapeDtypeStruct((N, D), jnp.float32),
    )(sums, cnts3, dst_embedding)
    return out
